# baseline (speedup 1.0000x reference)
import jax
import jax.numpy as jnp
from jax import lax
from jax.experimental import pallas as pl
from jax.experimental.pallas import tpu as pltpu
from jax.experimental.pallas import tpu_sc as plsc

_TRIM_FRACTION = 0.25
_ANOMALY_THRESHOLD = 0.9


def _dist_kernel(x_ref, g_ref, out_ref):
    x = x_ref[...]
    g = g_ref[...]
    d = x - g
    out_ref[...] = jnp.sum(d * d, axis=1).reshape(1, 1, -1)


def _rank_lower_median(v):
    n = v.shape[0]
    col = v[:, None]
    row = v[None, :]
    ii = lax.broadcasted_iota(jnp.int32, (n, n), 0)
    jj = lax.broadcasted_iota(jnp.int32, (n, n), 1)
    m = (row < col) | ((row == col) & (jj < ii))
    rank = jnp.sum(m.astype(jnp.int32), axis=1)
    return jnp.sum(jnp.where(rank == (n - 1) // 2, v, 0.0))


def _agg_kernel(partials_ref, x_ref, out_ref, keep_ref, scal_ref):
    i = pl.program_id(0)
    n = x_ref.shape[0]

    @pl.when(i == 0)
    def _():
        dist2 = jnp.sum(partials_ref[...], axis=(0, 1))
        d = jnp.sqrt(dist2)
        med = _rank_lower_median(d)
        mad = _rank_lower_median(jnp.abs(d - med))
        thr = med + _ANOMALY_THRESHOLD * mad
        keep = (d <= thr).astype(jnp.float32)
        k = jnp.sum(keep)
        none_kept = k == 0.0
        keep_eff = jnp.where(none_kept, jnp.ones_like(keep), keep)
        k_eff = jnp.where(none_kept, jnp.float32(n), k)
        t = jnp.floor(k_eff * _TRIM_FRACTION)
        keep_ref[...] = keep_eff[:, None]
        scal_ref[0] = t
        scal_ref[1] = k_eff - t
        scal_ref[2] = k_eff - 2.0 * t

    x = x_ref[...]
    keep = keep_ref[...]
    xm = jnp.where(keep > 0.0, x, jnp.inf)
    t = scal_ref[0]
    kmt = scal_ref[1]
    count = scal_ref[2]

    rows = lax.broadcasted_iota(jnp.int32, (n, 1), 0)
    pos = rows.astype(jnp.float32)
    sel = (pos >= t) & (pos < kmt)

    def _cmpx(blk, d, h, asc):
        rr = lax.broadcasted_iota(jnp.int32, (h, 1), 0)
        bd = (rr & d) != 0
        down = pltpu.roll(blk, d, axis=0)
        upv = pltpu.roll(blk, h - d, axis=0)
        if asc:
            return jnp.where(bd, jnp.maximum(blk, down),
                             jnp.minimum(blk, upv))
        return jnp.where(bd, jnp.minimum(blk, down),
                         jnp.maximum(blk, upv))

    def _sort_net(xs):
        for k in (2, 4):
            d = k // 2
            while d >= 1:
                bit_d = (rows & d) != 0
                bit_k = (rows & k) != 0
                down = pltpu.roll(xs, d, axis=0)
                upv = pltpu.roll(xs, n - d, axis=0)
                p = jnp.where(bit_d, down, upv)
                mn = jnp.minimum(xs, p)
                mx = jnp.maximum(xs, p)
                keep_min = jnp.logical_not(jnp.logical_xor(bit_d, bit_k))
                xs = jnp.where(keep_min, mn, mx)
                d //= 2
        k = 8
        while k <= n:
            d = k // 2
            while d >= 1:
                if k == n:
                    xs = _cmpx(xs, d, n, True)
                else:
                    xs = jnp.concatenate(
                        [_cmpx(xs[m * k:(m + 1) * k], d, k, m % 2 == 0)
                         for m in range(n // k)], axis=0)
                d //= 2
            k *= 2
        return xs

    sub = 128
    b = x.shape[1]
    outs = []
    for c0 in range(0, b, sub):
        xs = _sort_net(xm[:, c0:c0 + sub])
        outs.append(jnp.sum(jnp.where(sel, xs, 0.0), axis=0))
    s = jnp.concatenate(outs)
    out_ref[...] = (s / count).reshape(1, 1, -1)


_SC_W = 1024
_SC_NW = 32
_SC_N = 64


def _sc_net_pairs(n):
    pairs = []
    k = 2
    while k <= n:
        d = k // 2
        while d >= 1:
            for i in range(n):
                j = i ^ d
                if j > i:
                    asc = True if k == n else (i & k) == 0
                    pairs.append((i, j, asc))
            d //= 2
        k *= 2
    return pairs


_SC_PAIRS = _sc_net_pairs(_SC_N)


def _sc_body_fn(c_sc, grp_off):
    ngrp = _SC_W // 16

    def body(x_hbm, meta_hbm, out_hbm, xb, mb, ob):
        wid = lax.axis_index("s") * 2 + lax.axis_index("c")
        cols_per_w = c_sc // _SC_NW
        nchunks = cols_per_w // _SC_W
        base_grp = wid * (cols_per_w // 16)

        pltpu.sync_copy(meta_hbm, mb)
        t_v = mb[_SC_N]
        kmt_v = mb[_SC_N + 1]
        inv_v = mb[_SC_N + 3]
        infv = jnp.full((16,), jnp.inf, jnp.float32)
        zerov = jnp.zeros((16,), jnp.float32)

        def chunk_body(ci, _):
            rel0 = (base_grp + ci * ngrp) * 16
            pltpu.sync_copy(
                x_hbm.at[:, pl.ds(grp_off * 16 + rel0, _SC_W)], xb)

            def grp_body(g, _):
                c16 = g * 16
                vals = [jnp.where(mb[i] > 0.0, xb[i, pl.ds(c16, 16)], infv)
                        for i in range(_SC_N)]
                for i, j, asc in _SC_PAIRS:
                    a, b = vals[i], vals[j]
                    lo = jnp.minimum(a, b)
                    hi = jnp.maximum(a, b)
                    vals[i], vals[j] = (lo, hi) if asc else (hi, lo)
                acc = zerov
                for i in range(_SC_N):
                    fi = jnp.full((16,), jnp.float32(i))
                    m = (fi >= t_v) & (fi < kmt_v)
                    acc = acc + jnp.where(m, vals[i], zerov)
                ob[pl.ds(c16, 16)] = acc * inv_v
                return 0

            lax.fori_loop(0, ngrp, grp_body, 0)
            pltpu.sync_copy(ob, out_hbm.at[pl.ds(rel0, _SC_W)])
            return 0

        lax.fori_loop(0, nchunks, chunk_body, 0)

    return body


def _thresh_kernel(partials_ref, out_ref):
    dist2 = jnp.sum(partials_ref[...], axis=(0, 1))
    n = dist2.shape[0]
    d = jnp.sqrt(dist2)
    med = _rank_lower_median(d)
    mad = _rank_lower_median(jnp.abs(d - med))
    thr = med + _ANOMALY_THRESHOLD * mad
    keep = (d <= thr).astype(jnp.float32)
    k = jnp.sum(keep)
    none_kept = k == 0.0
    keep_eff = jnp.where(none_kept, jnp.ones_like(keep), keep)
    k_eff = jnp.where(none_kept, jnp.float32(n), k)
    t = jnp.floor(k_eff * _TRIM_FRACTION)
    kmt = k_eff - t
    count = k_eff - 2.0 * t
    out_ref[...] = jnp.concatenate(
        [jnp.broadcast_to(keep_eff[:, None], (n, 16)),
         jnp.broadcast_to(t, (1, 16)),
         jnp.broadcast_to(kmt, (1, 16)),
         jnp.broadcast_to(count, (1, 16)),
         jnp.broadcast_to(1.0 / count, (1, 16))], axis=0)


def _sc_pass2(x3, meta, c_sc, grp_off):
    run = pl.kernel(
        _sc_body_fn(c_sc, grp_off),
        mesh=plsc.VectorSubcoreMesh(core_axis_name="c", subcore_axis_name="s"),
        compiler_params=pltpu.CompilerParams(use_tc_tiling_on_sc=False),
        out_type=jax.ShapeDtypeStruct((c_sc,), jnp.float32),
        scratch_types=[
            pltpu.VMEM((_SC_N, _SC_W), jnp.float32),
            pltpu.VMEM((_SC_N + 4, 16), jnp.float32),
            pltpu.VMEM((_SC_W,), jnp.float32),
        ],
    )
    return run(x3, meta)


_SC_UNITS = 14


def kernel(updates, global_model):
    n, c = updates.shape
    block = 8192 if c % 8192 == 0 else 256
    nb = c // block
    g2d = global_model.reshape(1, c)

    partials = pl.pallas_call(
        _dist_kernel,
        grid=(nb,),
        in_specs=[
            pl.BlockSpec((n, block), lambda i: (0, i)),
            pl.BlockSpec((1, block), lambda i: (0, i)),
        ],
        out_specs=pl.BlockSpec((1, 1, n), lambda i: (i, 0, 0)),
        out_shape=jax.ShapeDtypeStruct((nb, 1, n), jnp.float32),
    )(updates, g2d)

    c_sc = min(_SC_UNITS, c // (_SC_NW * _SC_W)) * _SC_NW * _SC_W
    c_tc = c - c_sc
    nb_tc = c_tc // block

    outs = []
    if nb_tc:
        out3 = pl.pallas_call(
            _agg_kernel,
            grid=(nb_tc,),
            in_specs=[
                pl.BlockSpec((nb, 1, n), lambda i: (0, 0, 0)),
                pl.BlockSpec((n, block), lambda i: (0, i)),
            ],
            out_specs=pl.BlockSpec((1, 1, block), lambda i: (i, 0, 0)),
            out_shape=jax.ShapeDtypeStruct((nb_tc, 1, block), jnp.float32),
            scratch_shapes=[
                pltpu.VMEM((n, 1), jnp.float32),
                pltpu.SMEM((4,), jnp.float32),
            ],
        )(partials, updates)
        outs.append(out3.reshape(c_tc))

    if c_sc:
        meta = pl.pallas_call(
            _thresh_kernel,
            grid=(1,),
            in_specs=[pl.BlockSpec((nb, 1, n), lambda i: (0, 0, 0))],
            out_specs=pl.BlockSpec((n + 4, 16), lambda i: (0, 0)),
            out_shape=jax.ShapeDtypeStruct((n + 4, 16), jnp.float32),
        )(partials)
        outs.append(_sc_pass2(updates, meta, c_sc, c_tc // 16))

    return outs[0] if len(outs) == 1 else jnp.concatenate(outs)


# speedup vs baseline: 1.1060x; 1.1060x over previous
import jax
import jax.numpy as jnp
from jax import lax
from jax.experimental import pallas as pl
from jax.experimental.pallas import tpu as pltpu
from jax.experimental.pallas import tpu_sc as plsc

_TRIM_FRACTION = 0.25
_ANOMALY_THRESHOLD = 0.9


def _dist_kernel(x_ref, g_ref, out_ref):
    x = x_ref[...]
    g = g_ref[...]
    d = x - g
    out_ref[...] = jnp.sum(d * d, axis=1).reshape(1, 1, -1)


def _rank_lower_median(v):
    n = v.shape[0]
    col = v[:, None]
    row = v[None, :]
    ii = lax.broadcasted_iota(jnp.int32, (n, n), 0)
    jj = lax.broadcasted_iota(jnp.int32, (n, n), 1)
    m = (row < col) | ((row == col) & (jj < ii))
    rank = jnp.sum(m.astype(jnp.int32), axis=1)
    return jnp.sum(jnp.where(rank == (n - 1) // 2, v, 0.0))


def _agg_kernel(partials_ref, x_ref, out_ref, keep_ref, scal_ref):
    i = pl.program_id(0)
    n = x_ref.shape[0]

    @pl.when(i == 0)
    def _():
        dist2 = jnp.sum(partials_ref[...], axis=(0, 1))
        d = jnp.sqrt(dist2)
        med = _rank_lower_median(d)
        mad = _rank_lower_median(jnp.abs(d - med))
        thr = med + _ANOMALY_THRESHOLD * mad
        keep = (d <= thr).astype(jnp.float32)
        k = jnp.sum(keep)
        none_kept = k == 0.0
        keep_eff = jnp.where(none_kept, jnp.ones_like(keep), keep)
        k_eff = jnp.where(none_kept, jnp.float32(n), k)
        t = jnp.floor(k_eff * _TRIM_FRACTION)
        keep_ref[...] = keep_eff[:, None]
        scal_ref[0] = t
        scal_ref[1] = k_eff - t
        scal_ref[2] = k_eff - 2.0 * t

    x = x_ref[...]
    keep = keep_ref[...]
    xm = jnp.where(keep > 0.0, x, jnp.inf)
    t = scal_ref[0]
    kmt = scal_ref[1]
    count = scal_ref[2]

    rows = lax.broadcasted_iota(jnp.int32, (n, 1), 0)
    pos = rows.astype(jnp.float32)
    sel = (pos >= t) & (pos < kmt)

    def _cmpx(blk, d, h, asc):
        rr = lax.broadcasted_iota(jnp.int32, (h, 1), 0)
        bd = (rr & d) != 0
        down = pltpu.roll(blk, d, axis=0)
        upv = pltpu.roll(blk, h - d, axis=0)
        if asc:
            return jnp.where(bd, jnp.maximum(blk, down),
                             jnp.minimum(blk, upv))
        return jnp.where(bd, jnp.minimum(blk, down),
                         jnp.maximum(blk, upv))

    def _sort_net(xs):
        for k in (2, 4):
            d = k // 2
            while d >= 1:
                bit_d = (rows & d) != 0
                bit_k = (rows & k) != 0
                down = pltpu.roll(xs, d, axis=0)
                upv = pltpu.roll(xs, n - d, axis=0)
                p = jnp.where(bit_d, down, upv)
                mn = jnp.minimum(xs, p)
                mx = jnp.maximum(xs, p)
                keep_min = jnp.logical_not(jnp.logical_xor(bit_d, bit_k))
                xs = jnp.where(keep_min, mn, mx)
                d //= 2
        k = 8
        while k <= n:
            d = k // 2
            while d >= 1:
                if k == n:
                    xs = _cmpx(xs, d, n, True)
                else:
                    xs = jnp.concatenate(
                        [_cmpx(xs[m * k:(m + 1) * k], d, k, m % 2 == 0)
                         for m in range(n // k)], axis=0)
                d //= 2
            k *= 2
        return xs

    sub = 128
    b = x.shape[1]
    outs = []
    for c0 in range(0, b, sub):
        xs = _sort_net(xm[:, c0:c0 + sub])
        outs.append(jnp.sum(jnp.where(sel, xs, 0.0), axis=0))
    s = jnp.concatenate(outs)
    out_ref[...] = (s / count).reshape(1, 1, -1)


_SC_W = 1024
_SC_NW = 32
_SC_N = 64


def _sc_net_pairs(n):
    pairs = []
    k = 2
    while k <= n:
        d = k // 2
        while d >= 1:
            for i in range(n):
                j = i ^ d
                if j > i:
                    asc = True if k == n else (i & k) == 0
                    pairs.append((i, j, asc))
            d //= 2
        k *= 2
    return pairs


_SC_PAIRS = _sc_net_pairs(_SC_N)


def _sc_body_fn(c_sc, grp_off):
    ngrp = _SC_W // 16

    def body(x_hbm, meta_hbm, out_hbm, xb, mb, ob):
        wid = lax.axis_index("s") * 2 + lax.axis_index("c")
        cols_per_w = c_sc // _SC_NW
        nchunks = cols_per_w // _SC_W
        base_grp = wid * (cols_per_w // 16)

        pltpu.sync_copy(meta_hbm, mb)
        t_v = mb[_SC_N]
        kmt_v = mb[_SC_N + 1]
        inv_v = mb[_SC_N + 3]
        infv = jnp.full((16,), jnp.inf, jnp.float32)
        zerov = jnp.zeros((16,), jnp.float32)

        def chunk_body(ci, _):
            rel0 = (base_grp + ci * ngrp) * 16
            pltpu.sync_copy(
                x_hbm.at[:, pl.ds(grp_off * 16 + rel0, _SC_W)], xb)

            def grp_body(g, _):
                c16 = g * 16
                vals = [jnp.where(mb[i] > 0.0, xb[i, pl.ds(c16, 16)], infv)
                        for i in range(_SC_N)]
                for i, j, asc in _SC_PAIRS:
                    a, b = vals[i], vals[j]
                    lo = jnp.minimum(a, b)
                    hi = jnp.maximum(a, b)
                    vals[i], vals[j] = (lo, hi) if asc else (hi, lo)
                acc = zerov
                for i in range(_SC_N):
                    fi = jnp.full((16,), jnp.float32(i))
                    m = (fi >= t_v) & (fi < kmt_v)
                    acc = acc + jnp.where(m, vals[i], zerov)
                ob[pl.ds(c16, 16)] = acc * inv_v
                return 0

            lax.fori_loop(0, ngrp, grp_body, 0)
            pltpu.sync_copy(ob, out_hbm.at[pl.ds(rel0, _SC_W)])
            return 0

        lax.fori_loop(0, nchunks, chunk_body, 0)

    return body


def _thresh_kernel(partials_ref, out_ref):
    dist2 = jnp.sum(partials_ref[...], axis=(0, 1))
    n = dist2.shape[0]
    d = jnp.sqrt(dist2)
    med = _rank_lower_median(d)
    mad = _rank_lower_median(jnp.abs(d - med))
    thr = med + _ANOMALY_THRESHOLD * mad
    keep = (d <= thr).astype(jnp.float32)
    k = jnp.sum(keep)
    none_kept = k == 0.0
    keep_eff = jnp.where(none_kept, jnp.ones_like(keep), keep)
    k_eff = jnp.where(none_kept, jnp.float32(n), k)
    t = jnp.floor(k_eff * _TRIM_FRACTION)
    kmt = k_eff - t
    count = k_eff - 2.0 * t
    out_ref[...] = jnp.concatenate(
        [jnp.broadcast_to(keep_eff[:, None], (n, 16)),
         jnp.broadcast_to(t, (1, 16)),
         jnp.broadcast_to(kmt, (1, 16)),
         jnp.broadcast_to(count, (1, 16)),
         jnp.broadcast_to(1.0 / count, (1, 16))], axis=0)


def _sc_pass2(x3, meta, c_sc, grp_off):
    run = pl.kernel(
        _sc_body_fn(c_sc, grp_off),
        mesh=plsc.VectorSubcoreMesh(core_axis_name="c", subcore_axis_name="s"),
        compiler_params=pltpu.CompilerParams(use_tc_tiling_on_sc=False),
        out_type=jax.ShapeDtypeStruct((c_sc,), jnp.float32),
        scratch_types=[
            pltpu.VMEM((_SC_N, _SC_W), jnp.float32),
            pltpu.VMEM((_SC_N + 4, 16), jnp.float32),
            pltpu.VMEM((_SC_W,), jnp.float32),
        ],
    )
    return run(x3, meta)


_SC_UNITS = 18


def kernel(updates, global_model):
    n, c = updates.shape
    block = 8192 if c % 8192 == 0 else 256
    nb = c // block
    g2d = global_model.reshape(1, c)

    partials = pl.pallas_call(
        _dist_kernel,
        grid=(nb,),
        in_specs=[
            pl.BlockSpec((n, block), lambda i: (0, i)),
            pl.BlockSpec((1, block), lambda i: (0, i)),
        ],
        out_specs=pl.BlockSpec((1, 1, n), lambda i: (i, 0, 0)),
        out_shape=jax.ShapeDtypeStruct((nb, 1, n), jnp.float32),
    )(updates, g2d)

    c_sc = min(_SC_UNITS, c // (_SC_NW * _SC_W)) * _SC_NW * _SC_W
    c_tc = c - c_sc
    nb_tc = c_tc // block

    outs = []
    if nb_tc:
        out3 = pl.pallas_call(
            _agg_kernel,
            grid=(nb_tc,),
            in_specs=[
                pl.BlockSpec((nb, 1, n), lambda i: (0, 0, 0)),
                pl.BlockSpec((n, block), lambda i: (0, i)),
            ],
            out_specs=pl.BlockSpec((1, 1, block), lambda i: (i, 0, 0)),
            out_shape=jax.ShapeDtypeStruct((nb_tc, 1, block), jnp.float32),
            scratch_shapes=[
                pltpu.VMEM((n, 1), jnp.float32),
                pltpu.SMEM((4,), jnp.float32),
            ],
        )(partials, updates)
        outs.append(out3.reshape(c_tc))

    if c_sc:
        meta = pl.pallas_call(
            _thresh_kernel,
            grid=(1,),
            in_specs=[pl.BlockSpec((nb, 1, n), lambda i: (0, 0, 0))],
            out_specs=pl.BlockSpec((n + 4, 16), lambda i: (0, 0)),
            out_shape=jax.ShapeDtypeStruct((n + 4, 16), jnp.float32),
        )(partials)
        outs.append(_sc_pass2(updates, meta, c_sc, c_tc // 16))

    return outs[0] if len(outs) == 1 else jnp.concatenate(outs)
